# baseline (speedup 1.0000x reference)
import jax
import jax.numpy as jnp
from jax import lax
from jax.experimental import pallas as pl
from jax.experimental.pallas import tpu as pltpu
from jax.experimental.pallas import tpu_sc as plsc
from jax._src.pallas import mpmd as pl_mpmd

B, L, V, D = 4096, 200, 100000, 128
DE = 8
DO = D + 2 * DE
N_TOK = B * L

NC, NS = 2, 16
NW = NC * NS
TOK_PER_W = N_TOK // NW
K = 2
CHUNK = K * 128
N_CHUNKS = TOK_PER_W // CHUNK
ROWS_PER_W = TOK_PER_W // 128

N_ROWS = N_TOK // 128
TC_BLK = 8192


def _sc_body(x_hbm, w_hbm, en_hbm, out_hbm,
             idx0, word0, idx1, word1,
             sem_g, sem_out0, sem_out1, sem_idx0, sem_idx1):
    del en_hbm
    wid = lax.axis_index("s") * NC + lax.axis_index("c")
    tok0 = wid * TOK_PER_W
    row0 = wid * ROWS_PER_W

    slots = ((idx0, word0, sem_out0, sem_idx0),
             (idx1, word1, sem_out1, sem_idx1))

    def out_slice(c):
        base = tok0 + c * CHUNK
        return out_hbm.at[pl.ds(base, CHUNK), pl.ds(0, D)]

    def do_chunk(t, s):
        idx_v, word_v, sem_out, sem_idx = slots[s]
        other_idx, _, _, other_sem = slots[1 - s]
        c = t * 2 + s
        pltpu.make_async_copy(
            x_hbm.at[pl.ds(row0, K)], idx_v, sem_idx).wait()

        @pl.when(c < N_CHUNKS - 1)
        def _():
            r = row0 + (c + 1) * K
            pltpu.async_copy(x_hbm.at[pl.ds(r, K)], other_idx, other_sem)

        @pl.when(t >= 1)
        def _():
            pltpu.make_async_copy(word_v, out_slice(c), sem_out).wait()

        cps = [pltpu.async_copy(
            w_hbm.at[idx_v.at[j]], word_v.at[pl.ds(j * 128, 128)], sem_g)
            for j in range(K)]
        for cp in cps:
            cp.wait()
        pltpu.async_copy(word_v, out_slice(c), sem_out)

    pltpu.async_copy(x_hbm.at[pl.ds(row0, K)], idx0, sem_idx0)

    def outer(t, carry):
        do_chunk(t, 0)
        do_chunk(t, 1)
        return carry

    lax.fori_loop(0, N_CHUNKS // 2, outer, 0)

    for s in range(2):
        idx_v, word_v, sem_out, _ = slots[s]
        pltpu.make_async_copy(word_v, out_slice(N_CHUNKS - 2 + s),
                              sem_out).wait()


def _tc_body(xe_ref, xn_ref, went_ref, wneg_ref, out_ref, buf, sem):
    e = xe_ref[...]
    n = xn_ref[...]
    went = went_ref[...]
    wneg = wneg_ref[...]
    ent = jnp.zeros((DE, TC_BLK), jnp.float32)
    neg = jnp.zeros((DE, TC_BLK), jnp.float32)
    for r in range(3):
        ent = ent + jnp.where(e == r, went[:, r:r + 1], 0.0)
        neg = neg + jnp.where(n == r, wneg[:, r:r + 1], 0.0)
    buf[...] = jnp.concatenate([ent, neg], axis=0).T
    i = pl.program_id(0)
    pltpu.make_async_copy(
        buf, out_ref.at[pl.ds(i * TC_BLK, TC_BLK), pl.ds(D, 2 * DE)],
        sem).start()
    pltpu.make_async_copy(
        buf, out_ref.at[pl.ds(i * TC_BLK, TC_BLK), pl.ds(D, 2 * DE)],
        sem).wait()


@jax.jit
def _run(x2d, xe2d, xn2d, w, w_ent, w_neg):
    grid = (N_TOK // TC_BLK,)
    en_filled = pl.pallas_call(
        _tc_body,
        grid=grid,
        in_specs=[
            pl.BlockSpec((1, TC_BLK), lambda i: (0, i)),
            pl.BlockSpec((1, TC_BLK), lambda i: (0, i)),
            pl.BlockSpec((DE, 3), lambda i: (0, 0)),
            pl.BlockSpec((DE, 3), lambda i: (0, 0)),
        ],
        out_specs=pl.BlockSpec(memory_space=pl.ANY),
        out_shape=jax.ShapeDtypeStruct((N_TOK, DO), jnp.float32),
        scratch_shapes=[pltpu.VMEM((TC_BLK, 2 * DE), jnp.float32),
                        pltpu.SemaphoreType.DMA],
    )(xe2d, xn2d, w_ent, w_neg)

    mesh = plsc.VectorSubcoreMesh(core_axis_name="c", subcore_axis_name="s")
    sc = pl_mpmd._mpmd_map(
        [(mesh, _sc_body)],
        out_types=jax.ShapeDtypeStruct((N_TOK, DO), jnp.float32),
        input_output_aliases={2: 0},
        scratch_types=[
            pltpu.VMEM((K, 128), jnp.int32),
            pltpu.VMEM((CHUNK, D), jnp.float32),
            pltpu.VMEM((K, 128), jnp.int32),
            pltpu.VMEM((CHUNK, D), jnp.float32),
            pltpu.SemaphoreType.DMA,
            pltpu.SemaphoreType.DMA,
            pltpu.SemaphoreType.DMA,
            pltpu.SemaphoreType.DMA,
            pltpu.SemaphoreType.DMA,
        ],
        compiler_params=pltpu.CompilerParams(
            use_tc_tiling_on_sc=True, needs_layout_passes=False),
    )
    return sc(x2d, w, en_filled)


def kernel(x, x_entity, x_negation, target, text_inputs, use_elmo,
           W, W_entity, W_negation):
    out = _run(x.reshape(N_ROWS, 128).astype(jnp.int32),
               x_entity.reshape(1, N_TOK).astype(jnp.int32),
               x_negation.reshape(1, N_TOK).astype(jnp.int32),
               W, W_entity.T, W_negation.T)
    return out.reshape(B, L, DO)
